# Initial kernel scaffold; baseline (speedup 1.0000x reference)
import jax
import jax.numpy as jnp
from jax import lax
from jax.experimental import pallas as pl
from jax.experimental.pallas import tpu as pltpu
from jax.experimental.pallas import tpu_sc as plsc

N = 10000
E = 160000
DIN = 256
DOUT = 256
H = 4
NK = 2 * H
CH = DOUT // 2

NC = 2
NS = 16
L = 16

EP = 163840
E1_TILE = EP // 32
E1_CHUNKS = E1_TILE // 128
E2_TILE = EP // 16
E2_CHUNKS = E2_TILE // 32
R2 = 10016

BN = 1000
NBN = N // BN


def _iota16():
    return lax.iota(jnp.int32, L)


def _full16(v):
    return jnp.full((L,), v, jnp.int32)


def _mm_kernel(x_ref, w_ref, asr_ref, adr_ref, hg_ref, als_ref, ald_ref):
    k = pl.program_id(1)
    blk = jnp.dot(x_ref[...], w_ref[...], preferred_element_type=jnp.float32)
    hg_ref[...] = blk
    a_s = asr_ref[pl.ds(k, 1), :]
    a_d = adr_ref[pl.ds(k, 1), :]
    ps = jnp.sum(blk * a_s, axis=1)
    pd = jnp.sum(blk * a_d, axis=1)
    h = k // 2
    oh = (lax.broadcasted_iota(jnp.int32, (1, H), 1) == h).astype(jnp.float32)

    @pl.when(k == 0)
    def _():
        als_ref[...] = jnp.zeros_like(als_ref)
        ald_ref[...] = jnp.zeros_like(ald_ref)

    als_ref[...] += ps[:, None] * oh
    ald_ref[...] += pd[:, None] * oh


def _stage1(x, W, asr, adr):
    return pl.pallas_call(
        _mm_kernel,
        grid=(NBN, NK),
        in_specs=[
            pl.BlockSpec((BN, DIN), lambda nb, k: (nb, 0)),
            pl.BlockSpec((DIN, CH), lambda nb, k: (0, k)),
            pl.BlockSpec((NK, CH), lambda nb, k: (0, 0)),
            pl.BlockSpec((NK, CH), lambda nb, k: (0, 0)),
        ],
        out_specs=[
            pl.BlockSpec((BN, CH), lambda nb, k: (k * NBN + nb, 0)),
            pl.BlockSpec((BN, H), lambda nb, k: (nb, 0)),
            pl.BlockSpec((BN, H), lambda nb, k: (nb, 0)),
        ],
        out_shape=[
            jax.ShapeDtypeStruct((NK * N, CH), jnp.float32),
            jax.ShapeDtypeStruct((N, H), jnp.float32),
            jax.ShapeDtypeStruct((N, H), jnp.float32),
        ],
    )(x, W, asr, adr)


def _phase1_body(als_hbm, ald_hbm, src_hbm, dst_hbm, ev_hbm, den_hbm,
                 as_v, ad_v, src2, dst2, evp, den_acc):
    c = lax.axis_index("c")
    s = lax.axis_index("s")
    wid = c * NS + s
    base_e = wid * E1_TILE

    pltpu.sync_copy(als_hbm, as_v)
    pltpu.sync_copy(ald_hbm, ad_v)
    pltpu.sync_copy(src_hbm.at[pl.ds(wid * E1_CHUNKS, E1_CHUNKS)], src2)
    pltpu.sync_copy(dst_hbm.at[pl.ds(wid * E1_CHUNKS, E1_CHUNKS)], dst2)

    zero16 = jnp.zeros((L,), jnp.float32)

    @pl.loop(0, 128)
    def _(r):
        evp[r, :] = zero16

    @pl.loop(0, 5)
    def _(z):
        pltpu.sync_copy(evp.at[pl.ds(0, 125)],
                        den_acc.at[pl.ds(s * 625 + z * 125, 125)])
    plsc.subcore_barrier()

    @pl.loop(0, E1_CHUNKS)
    def _(j):
        for l in range(8):
            sv = src2[j, pl.ds(l * 16, 16)]
            dv = dst2[j, pl.ds(l * 16, 16)]
            eid = base_e + j * 128 + l * 16 + _iota16()
            live = eid < E
            for h in range(H):
                a = (plsc.load_gather(as_v, [sv, _full16(h)])
                     + plsc.load_gather(ad_v, [dv, _full16(h)]))
                a = jnp.maximum(a, 0.2 * a)
                ev = jnp.where(live, jnp.exp(a), 0.0)
                plsc.store_scatter(evp, [l * 16 + _iota16(), _full16(h)], ev)
        pltpu.sync_copy(evp, ev_hbm.at[pl.ds(base_e + j * 128, 128)])
        pltpu.sync_copy(evp, den_acc.at[dst2.at[j]], add=True)

    plsc.subcore_barrier()
    pltpu.sync_copy(den_acc.at[pl.ds(s * 625, 625)],
                    den_hbm.at[c].at[pl.ds(s * 625, 625)])


def _phase1(als, ald, src128, dst128):
    mesh = plsc.VectorSubcoreMesh(core_axis_name="c", subcore_axis_name="s")
    return pl.kernel(
        _phase1_body,
        out_type=[
            jax.ShapeDtypeStruct((EP, 16), jnp.float32),
            jax.ShapeDtypeStruct((NC, N, 16), jnp.float32),
        ],
        mesh=mesh,
        scratch_types=[
            pltpu.VMEM((N, H), jnp.float32),
            pltpu.VMEM((N, H), jnp.float32),
            pltpu.VMEM((E1_CHUNKS, 128), jnp.int32),
            pltpu.VMEM((E1_CHUNKS, 128), jnp.int32),
            pltpu.VMEM((128, 16), jnp.float32),
            pltpu.VMEM_SHARED((N, 16), jnp.float32),
        ],
    )(als, ald, src128, dst128)


def _rden_kernel(den_ref, rden_ref):
    p = den_ref[0, :, 0:H] + den_ref[1, :, 0:H]
    rden_ref[...] = 0.25 / (p + 1e-16)


def _stage3(denomp):
    return pl.pallas_call(
        _rden_kernel,
        in_specs=[pl.BlockSpec((NC, N, 16), lambda: (0, 0, 0))],
        out_specs=pl.BlockSpec((N, H), lambda: (0, 0)),
        out_shape=jax.ShapeDtypeStruct((N, H), jnp.float32),
    )(denomp)


def _phase2_body(hg_hbm, ev_hbm, rden_hbm, src_hbm, dst_hbm, oc_hbm,
                 rden_v, src32, dst32, evc, coefb, idxb, grows, accb, out_acc):
    c = lax.axis_index("c")
    s = lax.axis_index("s")
    base_e = s * E2_TILE

    pltpu.sync_copy(rden_hbm, rden_v)
    pltpu.sync_copy(src_hbm.at[pl.ds(s * E2_CHUNKS, E2_CHUNKS)], src32)
    pltpu.sync_copy(dst_hbm.at[pl.ds(s * E2_CHUNKS, E2_CHUNKS)], dst32)

    zero16 = jnp.zeros((L,), jnp.float32)

    @pl.loop(0, 128)
    def _(r):
        for q in range(8):
            grows[r, pl.ds(q * 16, 16)] = zero16

    @pl.loop(0, 4)
    def _(z):
        pltpu.sync_copy(grows, out_acc.at[pl.ds(s * 626 + z * 128, 128)])
    pltpu.sync_copy(grows.at[pl.ds(0, 114)],
                    out_acc.at[pl.ds(s * 626 + 512, 114)])
    plsc.subcore_barrier()

    @pl.loop(0, E2_CHUNKS)
    def _(ch):
        for h in range(H):
            for l in range(2):
                sv = src32[ch, pl.ds(l * 16, 16)]
                gi = sv + (2 * h) * N + c * N
                plsc.store_scatter(idxb, [_full16(0),
                                          h * 32 + l * 16 + _iota16()], gi)
        pltpu.sync_copy(hg_hbm.at[idxb.at[0]], grows)
        pltpu.sync_copy(ev_hbm.at[pl.ds(base_e + ch * 32, 32)], evc)
        for h in range(H):
            for l in range(2):
                dv = dst32[ch, pl.ds(l * 16, 16)]
                evh = plsc.load_gather(evc, [l * 16 + _iota16(), _full16(h)])
                rd = plsc.load_gather(rden_v, [dv, _full16(h)])
                plsc.store_scatter(coefb, [l * 16 + _iota16(), _full16(h)],
                                   evh * rd)

        @pl.loop(0, 32)
        def _(i):
            acc = [None] * 8
            for h in range(H):
                cb = plsc.load_gather(coefb, [jnp.full((L,), i, jnp.int32),
                                              _full16(h)])
                for q in range(8):
                    r = grows[h * 32 + i, pl.ds(q * 16, 16)]
                    acc[q] = r * cb if h == 0 else acc[q] + r * cb
            for q in range(8):
                accb[i, pl.ds(q * 16, 16)] = acc[q]

        pltpu.sync_copy(accb, out_acc.at[dst32.at[ch]], add=True)

    plsc.subcore_barrier()
    pltpu.sync_copy(out_acc.at[pl.ds(s * 626, 626)],
                    oc_hbm.at[c].at[pl.ds(s * 626, 626)])


def _phase2(hg, ev, rden, src32, dst32):
    mesh = plsc.VectorSubcoreMesh(core_axis_name="c", subcore_axis_name="s")
    return pl.kernel(
        _phase2_body,
        out_type=jax.ShapeDtypeStruct((NC, R2, CH), jnp.float32),
        mesh=mesh,
        scratch_types=[
            pltpu.VMEM((N, H), jnp.float32),
            pltpu.VMEM((E2_CHUNKS, 32), jnp.int32),
            pltpu.VMEM((E2_CHUNKS, 32), jnp.int32),
            pltpu.VMEM((32, 16), jnp.float32),
            pltpu.VMEM((32, H), jnp.float32),
            pltpu.VMEM((1, 128), jnp.int32),
            pltpu.VMEM((128, CH), jnp.float32),
            pltpu.VMEM((32, CH), jnp.float32),
            pltpu.VMEM_SHARED((R2, CH), jnp.float32),
        ],
    )(hg, ev, rden, src32, dst32)


def _final_kernel(oc_ref, b_ref, out_ref):
    out_ref[...] = (jnp.concatenate([oc_ref[0], oc_ref[1]], axis=1)
                    + b_ref[...])


def _stage5(oc, b2):
    return pl.pallas_call(
        _final_kernel,
        grid=(NBN,),
        in_specs=[
            pl.BlockSpec((NC, BN, CH), lambda nb: (0, nb, 0)),
            pl.BlockSpec((1, DOUT), lambda nb: (0, 0)),
        ],
        out_specs=pl.BlockSpec((BN, DOUT), lambda nb: (nb, 0)),
        out_shape=jax.ShapeDtypeStruct((N, DOUT), jnp.float32),
    )(oc, b2)


def kernel(x, edge_index, W, a_src, a_dst, b):
    src = edge_index[0]
    dst = edge_index[1]
    pad = EP - E
    srcp = jnp.concatenate([src, jnp.zeros((pad,), jnp.int32)])
    dstp = jnp.concatenate([dst, jnp.zeros((pad,), jnp.int32)])
    asr = a_src.reshape(NK, CH)
    adr = a_dst.reshape(NK, CH)

    hg, als, ald = _stage1(x, W, asr, adr)
    ev, denomp = _phase1(als, ald,
                         srcp.reshape(EP // 128, 128),
                         dstp.reshape(EP // 128, 128))
    rden = _stage3(denomp)
    oc = _phase2(hg, ev, rden,
                 srcp.reshape(EP // 32, 32),
                 dstp.reshape(EP // 32, 32))
    return _stage5(oc, b.reshape(1, DOUT))


# speedup vs baseline: 8.7295x; 8.7295x over previous
import dataclasses

import jax
import jax.numpy as jnp
from jax import lax
from jax.experimental import pallas as pl
from jax.experimental.pallas import tpu as pltpu
from jax.experimental.pallas import tpu_sc as plsc

N = 10000
E = 160000
DIN = 256
DOUT = 256
H = 4
NK = 2 * H
CH = DOUT // 2

NC = 2
NS = 16
L = 16

EP = 163840
ER = EP // 128
EVR = EP * H // 128
NH = 5120
RN = 10240

BN = 1000
NBN = N // BN


def _sc_params():
    cp = pltpu.CompilerParams()
    if "needs_layout_passes" in pltpu.CompilerParams.__dataclass_fields__:
        cp = dataclasses.replace(cp, needs_layout_passes=False)
    return cp


def _sc_mesh():
    return plsc.VectorSubcoreMesh(core_axis_name="c", subcore_axis_name="s",
                                  num_cores=NC, num_subcores=NS)


def _iota16():
    return lax.iota(jnp.int32, L)


def _full16(v):
    return jnp.full((L,), v, jnp.int32)


def _mm_kernel(x_ref, w_ref, asr_ref, adr_ref, hg_ref, alx_ref):
    k = pl.program_id(1)
    blk = jnp.dot(x_ref[...], w_ref[...], preferred_element_type=jnp.float32)
    hg_ref[...] = blk
    a_s = asr_ref[pl.ds(k, 1), :]
    a_d = adr_ref[pl.ds(k, 1), :]
    ps = jnp.sum(blk * a_s, axis=1)
    pd = jnp.sum(blk * a_d, axis=1)
    h = k // 2
    lanes = lax.broadcasted_iota(jnp.int32, (1, 128), 1)
    m_s = (lanes == h).astype(jnp.float32)
    m_d = (lanes == h + H).astype(jnp.float32)

    @pl.when(k == 0)
    def _():
        alx_ref[...] = jnp.zeros_like(alx_ref)

    alx_ref[...] += ps[:, None] * m_s + pd[:, None] * m_d


def _stage1(x, W, asr, adr):
    return pl.pallas_call(
        _mm_kernel,
        grid=(NBN, NK),
        in_specs=[
            pl.BlockSpec((BN, DIN), lambda nb, k: (nb, 0)),
            pl.BlockSpec((DIN, CH), lambda nb, k: (0, k)),
            pl.BlockSpec((NK, CH), lambda nb, k: (0, 0)),
            pl.BlockSpec((NK, CH), lambda nb, k: (0, 0)),
        ],
        out_specs=[
            pl.BlockSpec((BN, CH), lambda nb, k: (k * NBN + nb, 0)),
            pl.BlockSpec((BN, 128), lambda nb, k: (nb, 0)),
        ],
        out_shape=[
            jax.ShapeDtypeStruct((NK * N, CH), jnp.float32),
            jax.ShapeDtypeStruct((N, 128), jnp.float32),
        ],
    )(x, W, asr, adr)


def _phase1_body(alx_hbm, src_hbm, dst_hbm, ev_hbm, den_hbm,
                 srcc, dstc, alsb, aldb, evp, evpk, idxs, den_acc):
    c = lax.axis_index("c")
    s = lax.axis_index("s")
    nlo = c * NH

    zero16 = jnp.zeros((L,), jnp.float32)

    @pl.loop(0, 64)
    def _(r):
        for q in range(8):
            evp[r, pl.ds(q * 16, 16)] = zero16

    @pl.loop(0, 5)
    def _(z):
        for l in range(4):
            plsc.store_scatter(idxs, [_full16(0), l * 16 + _iota16()],
                               s * 320 + z * 64 + l * 16 + _iota16())
        pltpu.sync_copy(evp, den_acc.at[idxs.at[0]])
    plsc.subcore_barrier()

    @pl.loop(0, 10)
    def _(g):
        pltpu.sync_copy(src_hbm.at[pl.ds(s * 80 + g * 8, 8)], srcc)
        pltpu.sync_copy(dst_hbm.at[pl.ds(s * 80 + g * 8, 8)], dstc)

        @pl.loop(0, 4)
        def _(w):
            for half in range(2):
                row = w * 2 + half
                pltpu.sync_copy(alx_hbm.at[srcc.at[row]], alsb)
                pltpu.sync_copy(alx_hbm.at[dstc.at[row]], aldb)
                for sub in range(2):
                    for l4 in range(4):
                        dv = plsc.load_gather(
                            dstc, [jnp.full((L,), row, jnp.int32),
                                   sub * 64 + l4 * 16 + _iota16()])
                        dvl = dv - nlo
                        safe = (dvl >= 0) & (dvl < NH)
                        plsc.store_scatter(
                            idxs, [_full16(0), l4 * 16 + _iota16()],
                            jnp.where(safe, dvl, 0))
                        eid = (s * 10240 + g * 1024 + w * 256 + half * 128
                               + sub * 64 + l4 * 16 + _iota16())
                        live = (eid < E) & safe
                        for h in range(H):
                            er = sub * 64 + l4 * 16 + _iota16()
                            a = (plsc.load_gather(alsb, [er, _full16(h)])
                                 + plsc.load_gather(aldb,
                                                    [er, _full16(H + h)]))
                            a = jnp.maximum(a, 0.2 * a)
                            ev = jnp.exp(a)
                            plsc.store_scatter(
                                evp, [l4 * 16 + _iota16(), _full16(h)],
                                jnp.where(live, ev, 0.0))
                            plsc.store_scatter(
                                evpk,
                                [_full16(half * 4 + sub * 2 + (l4 >> 1)),
                                 _iota16() * 4 + ((l4 & 1) * 64 + h)],
                                jnp.where(eid < E, ev, 0.0))
                    pltpu.sync_copy(evp, den_acc.at[idxs.at[0]], add=True)

            @pl.when(c == 0)
            def _():
                pltpu.sync_copy(
                    evpk, ev_hbm.at[pl.ds(s * 320 + g * 32 + w * 8, 8)])

    plsc.subcore_barrier()
    pltpu.sync_copy(den_acc.at[pl.ds(s * 320, 320)],
                    den_hbm.at[c].at[pl.ds(s * 320, 320)])


def _phase1(alx, src128, dst128):
    return pl.kernel(
        _phase1_body,
        out_type=[
            jax.ShapeDtypeStruct((EVR, 128), jnp.float32),
            jax.ShapeDtypeStruct((NC, NH, 128), jnp.float32),
        ],
        mesh=_sc_mesh(),
        compiler_params=_sc_params(),
        scratch_types=[
            pltpu.VMEM((8, 128), jnp.int32),
            pltpu.VMEM((8, 128), jnp.int32),
            pltpu.VMEM((128, 128), jnp.float32),
            pltpu.VMEM((128, 128), jnp.float32),
            pltpu.VMEM((64, 128), jnp.float32),
            pltpu.VMEM((8, 128), jnp.float32),
            pltpu.VMEM((1, 64), jnp.int32),
            pltpu.VMEM_SHARED((NH, 128), jnp.float32),
        ],
    )(alx, src128, dst128)


def _rden_kernel(den_ref, rden_ref):
    rden_ref[...] = 0.25 / (den_ref[...] + 1e-16)


def _stage3(denomp):
    return pl.pallas_call(
        _rden_kernel,
        in_specs=[pl.BlockSpec((NC, NH, 128), lambda: (0, 0, 0))],
        out_specs=pl.BlockSpec((NC, NH, 128), lambda: (0, 0, 0)),
        out_shape=jax.ShapeDtypeStruct((NC, NH, 128), jnp.float32),
    )(denomp)


def _coef_body(ev_hbm, rden_hbm, dst_hbm, coef_hbm, dstc, rdb, evk, coefk):
    c = lax.axis_index("c")
    s = lax.axis_index("s")
    wid = c * NS + s

    @pl.loop(0, 5)
    def _(g):
        pltpu.sync_copy(dst_hbm.at[pl.ds(wid * 40 + g * 8, 8)], dstc)

        @pl.loop(0, 4)
        def _(w):
            pltpu.sync_copy(
                ev_hbm.at[pl.ds(wid * 160 + g * 32 + w * 8, 8)], evk)
            for half in range(2):
                pltpu.sync_copy(rden_hbm.at[dstc.at[w * 2 + half]], rdb)
                for l2 in range(8):
                    l = half * 8 + l2
                    for h in range(H):
                        rd = plsc.load_gather(
                            rdb, [l2 * 16 + _iota16(), _full16(h)])
                        lane = _iota16() * 4 + ((l & 1) * 64 + h)
                        evv = plsc.load_gather(evk, [_full16(l >> 1), lane])
                        plsc.store_scatter(coefk, [_full16(l >> 1), lane],
                                           evv * rd)
            pltpu.sync_copy(
                coefk, coef_hbm.at[pl.ds(wid * 160 + g * 32 + w * 8, 8)])


def _stage3b(ev, rden2d, dst128):
    return pl.kernel(
        _coef_body,
        out_type=jax.ShapeDtypeStruct((EVR, 128), jnp.float32),
        mesh=_sc_mesh(),
        compiler_params=_sc_params(),
        scratch_types=[
            pltpu.VMEM((8, 128), jnp.int32),
            pltpu.VMEM((128, 128), jnp.float32),
            pltpu.VMEM((8, 128), jnp.float32),
            pltpu.VMEM((8, 128), jnp.float32),
        ],
    )(ev, rden2d, dst128)


def _phase2_body(hg_hbm, coef_hbm, src_hbm, dst_hbm, oc_hbm,
                 srcc, dstc, coefc, idxg, idxs, idxz, grows, accb, out_acc):
    c = lax.axis_index("c")
    s = lax.axis_index("s")

    zero16 = jnp.zeros((L,), jnp.float32)

    @pl.loop(0, 64)
    def _(r):
        for q in range(8):
            accb[r, pl.ds(q * 16, 16)] = zero16

    @pl.loop(0, 10)
    def _(z):
        for l in range(4):
            plsc.store_scatter(idxz, [_full16(0), l * 16 + _iota16()],
                               s * 640 + z * 64 + l * 16 + _iota16())
        pltpu.sync_copy(accb, out_acc.at[idxz.at[0]])
    plsc.subcore_barrier()

    @pl.loop(0, 10)
    def _(g):
        pltpu.sync_copy(src_hbm.at[pl.ds(s * 80 + g * 8, 8)], srcc)
        pltpu.sync_copy(dst_hbm.at[pl.ds(s * 80 + g * 8, 8)], dstc)

        @pl.loop(0, 4)
        def _(w):
            pltpu.sync_copy(
                coef_hbm.at[pl.ds(s * 320 + g * 32 + w * 8, 8)], coefc)

            @pl.loop(0, 4)
            def _(t):
                for l4 in range(4):
                    r4 = t * 64 + l4 * 16
                    dv = plsc.load_gather(
                        dstc, [jnp.full((L,), w * 2 + r4 // 128, jnp.int32),
                               (r4 % 128) + _iota16()])
                    plsc.store_scatter(idxs, [_full16(0), l4 * 16 + _iota16()],
                                       dv)
                for grp in range(4):
                    r16 = t * 64 + grp * 16
                    sv = plsc.load_gather(
                        srcc, [jnp.full((L,), w * 2 + r16 // 128, jnp.int32),
                               (r16 % 128) + _iota16()])
                    for h in range(H):
                        plsc.store_scatter(
                            idxg, [_full16(0), h * 16 + _iota16()],
                            sv + ((2 * h) * N + c * N))
                    pltpu.sync_copy(hg_hbm.at[idxg.at[0]], grows)

                    @pl.loop(0, 16)
                    def _(i):
                        p = (t * 64 + grp * 16 + i) * 4
                        acc = [None] * 8
                        for h in range(H):
                            cb = plsc.load_gather(
                                coefc, [jnp.full((L,), (p + h) >> 7,
                                                 jnp.int32),
                                        jnp.full((L,), (p + h) & 127,
                                                 jnp.int32)])
                            for q in range(8):
                                r = grows[h * 16 + i, pl.ds(q * 16, 16)]
                                acc[q] = (r * cb if h == 0
                                          else acc[q] + r * cb)
                        for q in range(8):
                            accb[grp * 16 + i, pl.ds(q * 16, 16)] = acc[q]

                pltpu.sync_copy(accb, out_acc.at[idxs.at[0]], add=True)

    plsc.subcore_barrier()
    pltpu.sync_copy(out_acc.at[pl.ds(s * 640, 640)],
                    oc_hbm.at[c].at[pl.ds(s * 640, 640)])


def _phase2(hg, coef, src128, dst128):
    return pl.kernel(
        _phase2_body,
        out_type=jax.ShapeDtypeStruct((NC, RN, CH), jnp.float32),
        mesh=_sc_mesh(),
        compiler_params=_sc_params(),
        scratch_types=[
            pltpu.VMEM((8, 128), jnp.int32),
            pltpu.VMEM((8, 128), jnp.int32),
            pltpu.VMEM((8, 128), jnp.float32),
            pltpu.VMEM((1, 64), jnp.int32),
            pltpu.VMEM((1, 64), jnp.int32),
            pltpu.VMEM((1, 64), jnp.int32),
            pltpu.VMEM((64, CH), jnp.float32),
            pltpu.VMEM((64, CH), jnp.float32),
            pltpu.VMEM_SHARED((RN, CH), jnp.float32),
        ],
    )(hg, coef, src128, dst128)


def _final_kernel(oc_ref, b_ref, out_ref):
    out_ref[...] = (jnp.concatenate([oc_ref[0], oc_ref[1]], axis=1)
                    + b_ref[...])


def _stage5(oc, b2):
    return pl.pallas_call(
        _final_kernel,
        grid=(NBN,),
        in_specs=[
            pl.BlockSpec((NC, BN, CH), lambda nb: (0, nb, 0)),
            pl.BlockSpec((1, DOUT), lambda nb: (0, 0)),
        ],
        out_specs=pl.BlockSpec((BN, DOUT), lambda nb: (nb, 0)),
        out_shape=jax.ShapeDtypeStruct((N, DOUT), jnp.float32),
    )(oc, b2)


def kernel(x, edge_index, W, a_src, a_dst, b):
    src = edge_index[0]
    dst = edge_index[1]
    pad = EP - E
    srcp = jnp.concatenate([src, jnp.zeros((pad,), jnp.int32)]).reshape(ER, 128)
    dstp = jnp.concatenate([dst, jnp.zeros((pad,), jnp.int32)]).reshape(ER, 128)
    asr = a_src.reshape(NK, CH)
    adr = a_dst.reshape(NK, CH)

    hg, alx = _stage1(x, W, asr, adr)
    ev, denomp = _phase1(alx, srcp, dstp)
    rden = _stage3(denomp)
    coef = _stage3b(ev, rden.reshape(NC * NH, 128), dstp)
    oc = _phase2(hg, coef, srcp, dstp)
    return _stage5(oc, b.reshape(1, DOUT))


# speedup vs baseline: 10.8292x; 1.2405x over previous
import dataclasses

import jax
import jax.numpy as jnp
from jax import lax
from jax.experimental import pallas as pl
from jax.experimental.pallas import tpu as pltpu
from jax.experimental.pallas import tpu_sc as plsc

N = 10000
E = 160000
DIN = 256
DOUT = 256
H = 4
NK = 2 * H
CH = DOUT // 2

NC = 2
NS = 16
L = 16

EP = 163840
ER = EP // 128
EVR = EP * H // 128
NH = 5120
RN = 10240

BN = 1000
NBN = N // BN


def _sc_params():
    cp = pltpu.CompilerParams()
    if "needs_layout_passes" in pltpu.CompilerParams.__dataclass_fields__:
        cp = dataclasses.replace(cp, needs_layout_passes=False)
    return cp


def _sc_mesh():
    return plsc.VectorSubcoreMesh(core_axis_name="c", subcore_axis_name="s",
                                  num_cores=NC, num_subcores=NS)


def _iota16():
    return lax.iota(jnp.int32, L)


def _full16(v):
    return jnp.full((L,), v, jnp.int32)


def _mm_kernel(x_ref, w_ref, asr_ref, adr_ref, hg_ref, alx_ref):
    k = pl.program_id(1)
    blk = jnp.dot(x_ref[...], w_ref[...], preferred_element_type=jnp.float32)
    hg_ref[...] = blk
    a_s = asr_ref[pl.ds(k, 1), :]
    a_d = adr_ref[pl.ds(k, 1), :]
    ps = jnp.sum(blk * a_s, axis=1)
    pd = jnp.sum(blk * a_d, axis=1)
    h = k // 2
    lanes = lax.broadcasted_iota(jnp.int32, (1, 128), 1)
    m_s = (lanes == h).astype(jnp.float32)
    m_d = (lanes == h + H).astype(jnp.float32)

    @pl.when(k == 0)
    def _():
        alx_ref[...] = jnp.zeros_like(alx_ref)

    alx_ref[...] += ps[:, None] * m_s + pd[:, None] * m_d


def _stage1(x, W, asr, adr):
    return pl.pallas_call(
        _mm_kernel,
        grid=(NBN, NK),
        in_specs=[
            pl.BlockSpec((BN, DIN), lambda nb, k: (nb, 0)),
            pl.BlockSpec((DIN, CH), lambda nb, k: (0, k)),
            pl.BlockSpec((NK, CH), lambda nb, k: (0, 0)),
            pl.BlockSpec((NK, CH), lambda nb, k: (0, 0)),
        ],
        out_specs=[
            pl.BlockSpec((BN, CH), lambda nb, k: (k * NBN + nb, 0)),
            pl.BlockSpec((BN, 128), lambda nb, k: (nb, 0)),
        ],
        out_shape=[
            jax.ShapeDtypeStruct((NK * N, CH), jnp.float32),
            jax.ShapeDtypeStruct((N, 128), jnp.float32),
        ],
    )(x, W, asr, adr)


def _phase1_body(alx_hbm, src_hbm, dst_hbm, ev_hbm, den_hbm,
                 srcc, dstc, alsb, aldb, evp, evpk, idxs, semA, semB,
                 den_acc):
    c = lax.axis_index("c")
    s = lax.axis_index("s")
    nlo = c * NH

    zero16 = jnp.zeros((L,), jnp.float32)

    @pl.loop(0, 64)
    def _(r):
        for q in range(8):
            evp[r, pl.ds(q * 16, 16)] = zero16

    @pl.loop(0, 5)
    def _(z):
        for l in range(4):
            plsc.store_scatter(idxs, [_full16(0), l * 16 + _iota16()],
                               s * 320 + z * 64 + l * 16 + _iota16())
        pltpu.sync_copy(evp, den_acc.at[idxs.at[0]])
    plsc.subcore_barrier()

    @pl.loop(0, 10)
    def _(g):
        pltpu.sync_copy(src_hbm.at[pl.ds(s * 80 + g * 8, 8)], srcc)
        pltpu.sync_copy(dst_hbm.at[pl.ds(s * 80 + g * 8, 8)], dstc)

        @pl.loop(0, 4)
        def _(w):
            for half in range(2):
                row = w * 2 + half
                d1 = pltpu.async_copy(alx_hbm.at[srcc.at[row]], alsb, semA)
                d2 = pltpu.async_copy(alx_hbm.at[dstc.at[row]], aldb, semB)
                d1.wait()
                d2.wait()
                for sub in range(2):
                    for l4 in range(4):
                        dv = plsc.load_gather(
                            dstc, [jnp.full((L,), row, jnp.int32),
                                   sub * 64 + l4 * 16 + _iota16()])
                        dvl = dv - nlo
                        safe = (dvl >= 0) & (dvl < NH)
                        plsc.store_scatter(
                            idxs, [_full16(0), l4 * 16 + _iota16()],
                            jnp.where(safe, dvl, 0))
                        eid = (s * 10240 + g * 1024 + w * 256 + half * 128
                               + sub * 64 + l4 * 16 + _iota16())
                        live = (eid < E) & safe
                        for h in range(H):
                            er = sub * 64 + l4 * 16 + _iota16()
                            a = (plsc.load_gather(alsb, [er, _full16(h)])
                                 + plsc.load_gather(aldb,
                                                    [er, _full16(H + h)]))
                            a = jnp.maximum(a, 0.2 * a)
                            ev = jnp.exp(a)
                            plsc.store_scatter(
                                evp, [l4 * 16 + _iota16(), _full16(h)],
                                jnp.where(live, ev, 0.0))
                            plsc.store_scatter(
                                evpk,
                                [_full16(half * 4 + sub * 2 + (l4 >> 1)),
                                 _iota16() * 4 + ((l4 & 1) * 64 + h)],
                                jnp.where(eid < E, ev, 0.0))
                    pltpu.sync_copy(evp, den_acc.at[idxs.at[0]], add=True)

            @pl.when(c == 0)
            def _():
                pltpu.sync_copy(
                    evpk, ev_hbm.at[pl.ds(s * 320 + g * 32 + w * 8, 8)])

    plsc.subcore_barrier()
    pltpu.sync_copy(den_acc.at[pl.ds(s * 320, 320)],
                    den_hbm.at[c].at[pl.ds(s * 320, 320)])


def _phase1(alx, src128, dst128):
    return pl.kernel(
        _phase1_body,
        out_type=[
            jax.ShapeDtypeStruct((EVR, 128), jnp.float32),
            jax.ShapeDtypeStruct((NC, NH, 128), jnp.float32),
        ],
        mesh=_sc_mesh(),
        compiler_params=_sc_params(),
        scratch_types=[
            pltpu.VMEM((8, 128), jnp.int32),
            pltpu.VMEM((8, 128), jnp.int32),
            pltpu.VMEM((128, 128), jnp.float32),
            pltpu.VMEM((128, 128), jnp.float32),
            pltpu.VMEM((64, 128), jnp.float32),
            pltpu.VMEM((8, 128), jnp.float32),
            pltpu.VMEM((1, 64), jnp.int32),
            pltpu.SemaphoreType.DMA,
            pltpu.SemaphoreType.DMA,
            pltpu.VMEM_SHARED((NH, 128), jnp.float32),
        ],
    )(alx, src128, dst128)


def _rden_kernel(den_ref, rden_ref):
    rden_ref[...] = 0.25 / (den_ref[...] + 1e-16)


def _stage3(denomp):
    return pl.pallas_call(
        _rden_kernel,
        in_specs=[pl.BlockSpec((NC, NH, 128), lambda: (0, 0, 0))],
        out_specs=pl.BlockSpec((NC, NH, 128), lambda: (0, 0, 0)),
        out_shape=jax.ShapeDtypeStruct((NC, NH, 128), jnp.float32),
    )(denomp)


def _coef_body(ev_hbm, rden_hbm, dst_hbm, coef_hbm, dstc, rdb, evk, coefk):
    c = lax.axis_index("c")
    s = lax.axis_index("s")
    wid = c * NS + s

    @pl.loop(0, 5)
    def _(g):
        pltpu.sync_copy(dst_hbm.at[pl.ds(wid * 40 + g * 8, 8)], dstc)

        @pl.loop(0, 4)
        def _(w):
            pltpu.sync_copy(
                ev_hbm.at[pl.ds(wid * 160 + g * 32 + w * 8, 8)], evk)
            for half in range(2):
                pltpu.sync_copy(rden_hbm.at[dstc.at[w * 2 + half]], rdb)
                for l2 in range(8):
                    l = half * 8 + l2
                    for h in range(H):
                        rd = plsc.load_gather(
                            rdb, [l2 * 16 + _iota16(), _full16(h)])
                        lane = _iota16() * 4 + ((l & 1) * 64 + h)
                        evv = plsc.load_gather(evk, [_full16(l >> 1), lane])
                        plsc.store_scatter(coefk, [_full16(l >> 1), lane],
                                           evv * rd)
            pltpu.sync_copy(
                coefk, coef_hbm.at[pl.ds(wid * 160 + g * 32 + w * 8, 8)])


def _stage3b(ev, rden2d, dst128):
    return pl.kernel(
        _coef_body,
        out_type=jax.ShapeDtypeStruct((EVR, 128), jnp.float32),
        mesh=_sc_mesh(),
        compiler_params=_sc_params(),
        scratch_types=[
            pltpu.VMEM((8, 128), jnp.int32),
            pltpu.VMEM((128, 128), jnp.float32),
            pltpu.VMEM((8, 128), jnp.float32),
            pltpu.VMEM((8, 128), jnp.float32),
        ],
    )(ev, rden2d, dst128)


def _phase2_body(hg_hbm, coef_hbm, src_hbm, dst_hbm, oc_hbm,
                 srcc, dstc, coefc, igA, igB, idxs, idxz, grA, grB, accb,
                 semA, semB, out_acc):
    c = lax.axis_index("c")
    s = lax.axis_index("s")

    zero16 = jnp.zeros((L,), jnp.float32)

    @pl.loop(0, 64)
    def _(r):
        for q in range(8):
            accb[r, pl.ds(q * 16, 16)] = zero16

    @pl.loop(0, 10)
    def _(z):
        for l in range(4):
            plsc.store_scatter(idxz, [_full16(0), l * 16 + _iota16()],
                               s * 640 + z * 64 + l * 16 + _iota16())
        pltpu.sync_copy(accb, out_acc.at[idxz.at[0]])
    plsc.subcore_barrier()

    @pl.loop(0, 10)
    def _(g):
        pltpu.sync_copy(src_hbm.at[pl.ds(s * 80 + g * 8, 8)], srcc)
        pltpu.sync_copy(dst_hbm.at[pl.ds(s * 80 + g * 8, 8)], dstc)

        @pl.loop(0, 4)
        def _(w):
            pltpu.sync_copy(
                coef_hbm.at[pl.ds(s * 320 + g * 32 + w * 8, 8)], coefc)

            def build_idx(k, ibuf):
                for l2 in range(2):
                    r = k * 32 + l2 * 16
                    sv = plsc.load_gather(
                        srcc, [jnp.full((L,), w * 2 + r // 128, jnp.int32),
                               (r % 128) + _iota16()])
                    for h in range(H):
                        plsc.store_scatter(
                            ibuf, [_full16(0), h * 32 + l2 * 16 + _iota16()],
                            sv + ((2 * h) * N + c * N))

            bufs = [(grA, igA, semA), (grB, igB, semB)]
            build_idx(0, bufs[0][1])
            dcur = pltpu.async_copy(hg_hbm.at[bufs[0][1].at[0]],
                                    bufs[0][0], bufs[0][2])
            for k in range(8):
                gr, ig, _sem = bufs[k % 2]
                if k < 7:
                    gn, ign, semn = bufs[(k + 1) % 2]
                    build_idx(k + 1, ign)
                    dnext = pltpu.async_copy(hg_hbm.at[ign.at[0]], gn, semn)
                for l2 in range(2):
                    r = k * 32 + l2 * 16
                    dv = plsc.load_gather(
                        dstc, [jnp.full((L,), w * 2 + r // 128, jnp.int32),
                               (r % 128) + _iota16()])
                    plsc.store_scatter(
                        idxs,
                        [_full16(0), (k % 2) * 32 + l2 * 16 + _iota16()], dv)
                dcur.wait()

                @pl.loop(0, 32)
                def _(i):
                    p = (k * 32 + i) * 4
                    acc = [None] * 8
                    for h in range(H):
                        cb = plsc.load_gather(
                            coefc, [jnp.full((L,), (p + h) >> 7, jnp.int32),
                                    jnp.full((L,), (p + h) & 127, jnp.int32)])
                        for q in range(8):
                            r = gr[h * 32 + i, pl.ds(q * 16, 16)]
                            acc[q] = (r * cb if h == 0 else acc[q] + r * cb)
                    for q in range(8):
                        accb[(k % 2) * 32 + i, pl.ds(q * 16, 16)] = acc[q]

                if k % 2 == 1:
                    pltpu.sync_copy(accb, out_acc.at[idxs.at[0]], add=True)
                if k < 7:
                    dcur = dnext

    plsc.subcore_barrier()
    pltpu.sync_copy(out_acc.at[pl.ds(s * 640, 640)],
                    oc_hbm.at[c].at[pl.ds(s * 640, 640)])


def _phase2(hg, coef, src128, dst128):
    return pl.kernel(
        _phase2_body,
        out_type=jax.ShapeDtypeStruct((NC, RN, CH), jnp.float32),
        mesh=_sc_mesh(),
        compiler_params=_sc_params(),
        scratch_types=[
            pltpu.VMEM((8, 128), jnp.int32),
            pltpu.VMEM((8, 128), jnp.int32),
            pltpu.VMEM((8, 128), jnp.float32),
            pltpu.VMEM((1, 128), jnp.int32),
            pltpu.VMEM((1, 128), jnp.int32),
            pltpu.VMEM((1, 64), jnp.int32),
            pltpu.VMEM((1, 64), jnp.int32),
            pltpu.VMEM((128, CH), jnp.float32),
            pltpu.VMEM((128, CH), jnp.float32),
            pltpu.VMEM((64, CH), jnp.float32),
            pltpu.SemaphoreType.DMA,
            pltpu.SemaphoreType.DMA,
            pltpu.VMEM_SHARED((RN, CH), jnp.float32),
        ],
    )(hg, coef, src128, dst128)


def _final_kernel(oc_ref, b_ref, out_ref):
    out_ref[...] = (jnp.concatenate([oc_ref[0], oc_ref[1]], axis=1)
                    + b_ref[...])


def _stage5(oc, b2):
    return pl.pallas_call(
        _final_kernel,
        grid=(NBN,),
        in_specs=[
            pl.BlockSpec((NC, BN, CH), lambda nb: (0, nb, 0)),
            pl.BlockSpec((1, DOUT), lambda nb: (0, 0)),
        ],
        out_specs=pl.BlockSpec((BN, DOUT), lambda nb: (nb, 0)),
        out_shape=jax.ShapeDtypeStruct((N, DOUT), jnp.float32),
    )(oc, b2)


def kernel(x, edge_index, W, a_src, a_dst, b):
    src = edge_index[0]
    dst = edge_index[1]
    pad = EP - E
    srcp = jnp.concatenate([src, jnp.zeros((pad,), jnp.int32)]).reshape(ER, 128)
    dstp = jnp.concatenate([dst, jnp.zeros((pad,), jnp.int32)]).reshape(ER, 128)
    asr = a_src.reshape(NK, CH)
    adr = a_dst.reshape(NK, CH)

    hg, alx = _stage1(x, W, asr, adr)
    ev, denomp = _phase1(alx, srcp, dstp)
    rden = _stage3(denomp)
    coef = _stage3b(ev, rden.reshape(NC * NH, 128), dstp)
    oc = _phase2(hg, coef, srcp, dstp)
    return _stage5(oc, b.reshape(1, DOUT))


# speedup vs baseline: 11.2079x; 1.0350x over previous
import dataclasses

import jax
import jax.numpy as jnp
from jax import lax
from jax.experimental import pallas as pl
from jax.experimental.pallas import tpu as pltpu
from jax.experimental.pallas import tpu_sc as plsc

N = 10000
E = 160000
DIN = 256
DOUT = 256
H = 4
NK = 2 * H
CH = DOUT // 2

NC = 2
NS = 16
L = 16

EP = 163840
ER = EP // 128
EVR = EP * H // 128
NH = 5120
RN = 10240

BN = 1000
NBN = N // BN


def _sc_params():
    cp = pltpu.CompilerParams()
    if "needs_layout_passes" in pltpu.CompilerParams.__dataclass_fields__:
        cp = dataclasses.replace(cp, needs_layout_passes=False)
    return cp


def _sc_mesh():
    return plsc.VectorSubcoreMesh(core_axis_name="c", subcore_axis_name="s",
                                  num_cores=NC, num_subcores=NS)


def _iota16():
    return lax.iota(jnp.int32, L)


def _full16(v):
    return jnp.full((L,), v, jnp.int32)


def _mm_kernel(x_ref, w_ref, asr_ref, adr_ref, hg_ref, alx_ref):
    k = pl.program_id(1)
    blk = jnp.dot(x_ref[...], w_ref[...], preferred_element_type=jnp.float32)
    hg_ref[...] = blk
    a_s = asr_ref[pl.ds(k, 1), :]
    a_d = adr_ref[pl.ds(k, 1), :]
    ps = jnp.sum(blk * a_s, axis=1)
    pd = jnp.sum(blk * a_d, axis=1)
    h = k // 2
    lanes = lax.broadcasted_iota(jnp.int32, (1, 128), 1)
    m_s = (lanes == h).astype(jnp.float32)
    m_d = (lanes == h + H).astype(jnp.float32)

    @pl.when(k == 0)
    def _():
        alx_ref[...] = jnp.zeros_like(alx_ref)

    alx_ref[...] += ps[:, None] * m_s + pd[:, None] * m_d


def _stage1(x, W, asr, adr):
    return pl.pallas_call(
        _mm_kernel,
        grid=(NBN, NK),
        in_specs=[
            pl.BlockSpec((BN, DIN), lambda nb, k: (nb, 0)),
            pl.BlockSpec((DIN, CH), lambda nb, k: (0, k)),
            pl.BlockSpec((NK, CH), lambda nb, k: (0, 0)),
            pl.BlockSpec((NK, CH), lambda nb, k: (0, 0)),
        ],
        out_specs=[
            pl.BlockSpec((BN, CH), lambda nb, k: (k * NBN + nb, 0)),
            pl.BlockSpec((BN, 128), lambda nb, k: (nb, 0)),
        ],
        out_shape=[
            jax.ShapeDtypeStruct((NK * N, CH), jnp.float32),
            jax.ShapeDtypeStruct((N, 128), jnp.float32),
        ],
    )(x, W, asr, adr)


def _phase1_body(alx_hbm, src_hbm, dst_hbm, ev_hbm, den_hbm,
                 srcc, dstc, alsA, aldA, alsB, aldB, evp, evpk, idxs, idxz,
                 semA, semB, semC, semD, den_acc):
    c = lax.axis_index("c")
    s = lax.axis_index("s")
    nlo = c * NH

    zero16 = jnp.zeros((L,), jnp.float32)

    @pl.loop(0, 128)
    def _(r):
        for q in range(8):
            evp[r, pl.ds(q * 16, 16)] = zero16

    @pl.loop(0, 5)
    def _(z):
        for l in range(4):
            plsc.store_scatter(idxz, [_full16(0), l * 16 + _iota16()],
                               s * 320 + z * 64 + l * 16 + _iota16())
        pltpu.sync_copy(evp.at[pl.ds(0, 64)], den_acc.at[idxz.at[0]])
    plsc.subcore_barrier()

    buf_pair = [(alsA, aldA, semA, semB), (alsB, aldB, semC, semD)]

    @pl.loop(0, 10)
    def _(g):
        pltpu.sync_copy(src_hbm.at[pl.ds(s * 80 + g * 8, 8)], srcc)
        pltpu.sync_copy(dst_hbm.at[pl.ds(s * 80 + g * 8, 8)], dstc)

        def issue(hh, bp):
            return (pltpu.async_copy(alx_hbm.at[srcc.at[hh]], bp[0], bp[2]),
                    pltpu.async_copy(alx_hbm.at[dstc.at[hh]], bp[1], bp[3]))

        dcur = issue(0, buf_pair[0])
        for hh in range(8):
            als, ald = buf_pair[hh % 2][0], buf_pair[hh % 2][1]
            if hh < 7:
                dnext = issue(hh + 1, buf_pair[(hh + 1) % 2])
            dcur[0].wait()
            dcur[1].wait()
            for l4 in range(8):
                dv = plsc.load_gather(
                    dstc, [jnp.full((L,), hh, jnp.int32),
                           l4 * 16 + _iota16()])
                dvl = dv - nlo
                safe = (dvl >= 0) & (dvl < NH)
                plsc.store_scatter(
                    idxs, [_full16(0), l4 * 16 + _iota16()],
                    jnp.where(safe, dvl, 0))
                eid = (s * 10240 + g * 1024 + hh * 128
                       + l4 * 16 + _iota16())
                live = (eid < E) & safe
                er = l4 * 16 + _iota16()
                for h in range(H):
                    a = (plsc.load_gather(als, [er, _full16(h)])
                         + plsc.load_gather(ald, [er, _full16(H + h)]))
                    a = jnp.maximum(a, 0.2 * a)
                    ev = jnp.exp(a)
                    plsc.store_scatter(
                        evp, [l4 * 16 + _iota16(), _full16(h)],
                        jnp.where(live, ev, 0.0))
                    plsc.store_scatter(
                        evpk,
                        [_full16((hh % 2) * 4 + (l4 >> 1)),
                         _iota16() * 4 + ((l4 & 1) * 64 + h)],
                        jnp.where(eid < E, ev, 0.0))
            pltpu.sync_copy(evp, den_acc.at[idxs.at[0]], add=True)
            if hh % 2 == 1:
                w8 = (hh // 2) * 8

                @pl.when(c == 0)
                def _():
                    pltpu.sync_copy(
                        evpk,
                        ev_hbm.at[pl.ds(s * 320 + g * 32 + w8, 8)])
            if hh < 7:
                dcur = dnext

    plsc.subcore_barrier()
    pltpu.sync_copy(den_acc.at[pl.ds(s * 320, 320)],
                    den_hbm.at[c].at[pl.ds(s * 320, 320)])


def _phase1(alx, src128, dst128):
    return pl.kernel(
        _phase1_body,
        out_type=[
            jax.ShapeDtypeStruct((EVR, 128), jnp.float32),
            jax.ShapeDtypeStruct((NC, NH, 128), jnp.float32),
        ],
        mesh=_sc_mesh(),
        compiler_params=_sc_params(),
        scratch_types=[
            pltpu.VMEM((8, 128), jnp.int32),
            pltpu.VMEM((8, 128), jnp.int32),
            pltpu.VMEM((128, 128), jnp.float32),
            pltpu.VMEM((128, 128), jnp.float32),
            pltpu.VMEM((128, 128), jnp.float32),
            pltpu.VMEM((128, 128), jnp.float32),
            pltpu.VMEM((128, 128), jnp.float32),
            pltpu.VMEM((8, 128), jnp.float32),
            pltpu.VMEM((1, 128), jnp.int32),
            pltpu.VMEM((1, 64), jnp.int32),
            pltpu.SemaphoreType.DMA,
            pltpu.SemaphoreType.DMA,
            pltpu.SemaphoreType.DMA,
            pltpu.SemaphoreType.DMA,
            pltpu.VMEM_SHARED((NH, 128), jnp.float32),
        ],
    )(alx, src128, dst128)


def _rden_kernel(den_ref, rden_ref):
    rden_ref[...] = 0.25 / (den_ref[...] + 1e-16)


def _stage3(denomp):
    return pl.pallas_call(
        _rden_kernel,
        in_specs=[pl.BlockSpec((NC, NH, 128), lambda: (0, 0, 0))],
        out_specs=pl.BlockSpec((NC, NH, 128), lambda: (0, 0, 0)),
        out_shape=jax.ShapeDtypeStruct((NC, NH, 128), jnp.float32),
    )(denomp)


def _coef_body(ev_hbm, rden_hbm, dst_hbm, coef_hbm, dstc, rdb, evk, coefk):
    c = lax.axis_index("c")
    s = lax.axis_index("s")
    wid = c * NS + s

    @pl.loop(0, 5)
    def _(g):
        pltpu.sync_copy(dst_hbm.at[pl.ds(wid * 40 + g * 8, 8)], dstc)

        @pl.loop(0, 4)
        def _(w):
            pltpu.sync_copy(
                ev_hbm.at[pl.ds(wid * 160 + g * 32 + w * 8, 8)], evk)
            for half in range(2):
                pltpu.sync_copy(rden_hbm.at[dstc.at[w * 2 + half]], rdb)
                for l2 in range(8):
                    l = half * 8 + l2
                    for h in range(H):
                        rd = plsc.load_gather(
                            rdb, [l2 * 16 + _iota16(), _full16(h)])
                        lane = _iota16() * 4 + ((l & 1) * 64 + h)
                        evv = plsc.load_gather(evk, [_full16(l >> 1), lane])
                        plsc.store_scatter(coefk, [_full16(l >> 1), lane],
                                           evv * rd)
            pltpu.sync_copy(
                coefk, coef_hbm.at[pl.ds(wid * 160 + g * 32 + w * 8, 8)])


def _stage3b(ev, rden2d, dst128):
    return pl.kernel(
        _coef_body,
        out_type=jax.ShapeDtypeStruct((EVR, 128), jnp.float32),
        mesh=_sc_mesh(),
        compiler_params=_sc_params(),
        scratch_types=[
            pltpu.VMEM((8, 128), jnp.int32),
            pltpu.VMEM((128, 128), jnp.float32),
            pltpu.VMEM((8, 128), jnp.float32),
            pltpu.VMEM((8, 128), jnp.float32),
        ],
    )(ev, rden2d, dst128)


def _phase2_body(hg_hbm, coef_hbm, src_hbm, dst_hbm, oc_hbm,
                 srcc, dstc, coefc, igA, igB, idxs, idxz, grA, grB, accb,
                 semA, semB, out_acc):
    c = lax.axis_index("c")
    s = lax.axis_index("s")

    zero16 = jnp.zeros((L,), jnp.float32)

    @pl.loop(0, 64)
    def _(r):
        for q in range(8):
            accb[r, pl.ds(q * 16, 16)] = zero16

    @pl.loop(0, 10)
    def _(z):
        for l in range(4):
            plsc.store_scatter(idxz, [_full16(0), l * 16 + _iota16()],
                               s * 640 + z * 64 + l * 16 + _iota16())
        pltpu.sync_copy(accb, out_acc.at[idxz.at[0]])
    plsc.subcore_barrier()

    @pl.loop(0, 10)
    def _(g):
        pltpu.sync_copy(src_hbm.at[pl.ds(s * 80 + g * 8, 8)], srcc)
        pltpu.sync_copy(dst_hbm.at[pl.ds(s * 80 + g * 8, 8)], dstc)

        @pl.loop(0, 4)
        def _(w):
            pltpu.sync_copy(
                coef_hbm.at[pl.ds(s * 320 + g * 32 + w * 8, 8)], coefc)

            def build_idx(k, ibuf):
                for l2 in range(2):
                    r = k * 32 + l2 * 16
                    sv = plsc.load_gather(
                        srcc, [jnp.full((L,), w * 2 + r // 128, jnp.int32),
                               (r % 128) + _iota16()])
                    for h in range(H):
                        plsc.store_scatter(
                            ibuf, [_full16(0), h * 32 + l2 * 16 + _iota16()],
                            sv + ((2 * h) * N + c * N))

            bufs = [(grA, igA, semA), (grB, igB, semB)]
            build_idx(0, bufs[0][1])
            dcur = pltpu.async_copy(hg_hbm.at[bufs[0][1].at[0]],
                                    bufs[0][0], bufs[0][2])
            for k in range(8):
                gr, ig, _sem = bufs[k % 2]
                if k < 7:
                    gn, ign, semn = bufs[(k + 1) % 2]
                    build_idx(k + 1, ign)
                    dnext = pltpu.async_copy(hg_hbm.at[ign.at[0]], gn, semn)
                for l2 in range(2):
                    r = k * 32 + l2 * 16
                    dv = plsc.load_gather(
                        dstc, [jnp.full((L,), w * 2 + r // 128, jnp.int32),
                               (r % 128) + _iota16()])
                    plsc.store_scatter(
                        idxs,
                        [_full16(0), (k % 2) * 32 + l2 * 16 + _iota16()], dv)
                dcur.wait()

                @pl.loop(0, 32)
                def _(i):
                    p = (k * 32 + i) * 4
                    acc = [None] * 8
                    for h in range(H):
                        cb = plsc.load_gather(
                            coefc, [jnp.full((L,), (p + h) >> 7, jnp.int32),
                                    jnp.full((L,), (p + h) & 127, jnp.int32)])
                        for q in range(8):
                            r = gr[h * 32 + i, pl.ds(q * 16, 16)]
                            acc[q] = (r * cb if h == 0 else acc[q] + r * cb)
                    for q in range(8):
                        accb[(k % 2) * 32 + i, pl.ds(q * 16, 16)] = acc[q]

                if k % 2 == 1:
                    pltpu.sync_copy(accb, out_acc.at[idxs.at[0]], add=True)
                if k < 7:
                    dcur = dnext

    plsc.subcore_barrier()
    pltpu.sync_copy(out_acc.at[pl.ds(s * 640, 640)],
                    oc_hbm.at[c].at[pl.ds(s * 640, 640)])


def _phase2(hg, coef, src128, dst128):
    return pl.kernel(
        _phase2_body,
        out_type=jax.ShapeDtypeStruct((NC, RN, CH), jnp.float32),
        mesh=_sc_mesh(),
        compiler_params=_sc_params(),
        scratch_types=[
            pltpu.VMEM((8, 128), jnp.int32),
            pltpu.VMEM((8, 128), jnp.int32),
            pltpu.VMEM((8, 128), jnp.float32),
            pltpu.VMEM((1, 128), jnp.int32),
            pltpu.VMEM((1, 128), jnp.int32),
            pltpu.VMEM((1, 64), jnp.int32),
            pltpu.VMEM((1, 64), jnp.int32),
            pltpu.VMEM((128, CH), jnp.float32),
            pltpu.VMEM((128, CH), jnp.float32),
            pltpu.VMEM((64, CH), jnp.float32),
            pltpu.SemaphoreType.DMA,
            pltpu.SemaphoreType.DMA,
            pltpu.VMEM_SHARED((RN, CH), jnp.float32),
        ],
    )(hg, coef, src128, dst128)


def _final_kernel(oc_ref, b_ref, out_ref):
    out_ref[...] = (jnp.concatenate([oc_ref[0], oc_ref[1]], axis=1)
                    + b_ref[...])


def _stage5(oc, b2):
    return pl.pallas_call(
        _final_kernel,
        grid=(NBN,),
        in_specs=[
            pl.BlockSpec((NC, BN, CH), lambda nb: (0, nb, 0)),
            pl.BlockSpec((1, DOUT), lambda nb: (0, 0)),
        ],
        out_specs=pl.BlockSpec((BN, DOUT), lambda nb: (nb, 0)),
        out_shape=jax.ShapeDtypeStruct((N, DOUT), jnp.float32),
    )(oc, b2)


def kernel(x, edge_index, W, a_src, a_dst, b):
    src = edge_index[0]
    dst = edge_index[1]
    pad = EP - E
    srcp = jnp.concatenate([src, jnp.zeros((pad,), jnp.int32)]).reshape(ER, 128)
    dstp = jnp.concatenate([dst, jnp.zeros((pad,), jnp.int32)]).reshape(ER, 128)
    asr = a_src.reshape(NK, CH)
    adr = a_dst.reshape(NK, CH)

    hg, alx = _stage1(x, W, asr, adr)
    ev, denomp = _phase1(alx, srcp, dstp)
    rden = _stage3(denomp)
    coef = _stage3b(ev, rden.reshape(NC * NH, 128), dstp)
    oc = _phase2(hg, coef, srcp, dstp)
    return _stage5(oc, b.reshape(1, DOUT))


# speedup vs baseline: 11.5024x; 1.0263x over previous
import dataclasses

import jax
import jax.numpy as jnp
from jax import lax
from jax.experimental import pallas as pl
from jax.experimental.pallas import tpu as pltpu
from jax.experimental.pallas import tpu_sc as plsc

N = 10000
E = 160000
DIN = 256
DOUT = 256
H = 4
NK = 2 * H
CH = DOUT // 2

NC = 2
NS = 16
L = 16

EP = 163840
ER = EP // 128
EVR = EP * H // 128
NH = 5120
RN = 10240

BN = 1000
NBN = N // BN


def _sc_params():
    cp = pltpu.CompilerParams()
    if "needs_layout_passes" in pltpu.CompilerParams.__dataclass_fields__:
        cp = dataclasses.replace(cp, needs_layout_passes=False)
    return cp


def _sc_mesh():
    return plsc.VectorSubcoreMesh(core_axis_name="c", subcore_axis_name="s",
                                  num_cores=NC, num_subcores=NS)


def _iota16():
    return lax.iota(jnp.int32, L)


def _full16(v):
    return jnp.full((L,), v, jnp.int32)


def _mm_kernel(x_ref, w_ref, asr_ref, adr_ref, hg_ref, alx_ref):
    k = pl.program_id(1)
    blk = jnp.dot(x_ref[...], w_ref[...], preferred_element_type=jnp.float32)
    hg_ref[...] = blk
    a_s = asr_ref[pl.ds(k, 1), :]
    a_d = adr_ref[pl.ds(k, 1), :]
    ps = jnp.sum(blk * a_s, axis=1)
    pd = jnp.sum(blk * a_d, axis=1)
    h = k // 2
    lanes = lax.broadcasted_iota(jnp.int32, (1, 128), 1)
    m_s = (lanes == h).astype(jnp.float32)
    m_d = (lanes == h + H).astype(jnp.float32)

    @pl.when(k == 0)
    def _():
        alx_ref[...] = jnp.zeros_like(alx_ref)

    alx_ref[...] += ps[:, None] * m_s + pd[:, None] * m_d


def _stage1(x, W, asr, adr):
    return pl.pallas_call(
        _mm_kernel,
        grid=(NBN, NK),
        in_specs=[
            pl.BlockSpec((BN, DIN), lambda nb, k: (nb, 0)),
            pl.BlockSpec((DIN, CH), lambda nb, k: (0, k)),
            pl.BlockSpec((NK, CH), lambda nb, k: (0, 0)),
            pl.BlockSpec((NK, CH), lambda nb, k: (0, 0)),
        ],
        out_specs=[
            pl.BlockSpec((BN, CH), lambda nb, k: (k * NBN + nb, 0)),
            pl.BlockSpec((BN, 128), lambda nb, k: (nb, 0)),
        ],
        out_shape=[
            jax.ShapeDtypeStruct((NK * N, CH), jnp.float32),
            jax.ShapeDtypeStruct((N, 128), jnp.float32),
        ],
    )(x, W, asr, adr)


def _phase1_body(alx_hbm, src_hbm, dst_hbm, ev_hbm, den_hbm,
                 srcc, dstc, alsA, aldA, alsB, aldB, evp, evpk, idxs, idxz,
                 semA, semB, semC, semD, den_acc):
    c = lax.axis_index("c")
    s = lax.axis_index("s")
    nlo = c * NH

    zero16 = jnp.zeros((L,), jnp.float32)

    @pl.loop(0, 128)
    def _(r):
        for q in range(8):
            evp[r, pl.ds(q * 16, 16)] = zero16

    @pl.loop(0, 5)
    def _(z):
        for l in range(4):
            plsc.store_scatter(idxz, [_full16(0), l * 16 + _iota16()],
                               s * 320 + z * 64 + l * 16 + _iota16())
        pltpu.sync_copy(evp.at[pl.ds(0, 64)], den_acc.at[idxz.at[0]])
    plsc.subcore_barrier()

    buf_pair = [(alsA, aldA, semA, semB), (alsB, aldB, semC, semD)]

    @pl.loop(0, 10)
    def _(g):
        pltpu.sync_copy(src_hbm.at[pl.ds(s * 80 + g * 8, 8)], srcc)
        pltpu.sync_copy(dst_hbm.at[pl.ds(s * 80 + g * 8, 8)], dstc)

        def issue(hh, bp):
            return (pltpu.async_copy(alx_hbm.at[srcc.at[hh]], bp[0], bp[2]),
                    pltpu.async_copy(alx_hbm.at[dstc.at[hh]], bp[1], bp[3]))

        dcur = issue(0, buf_pair[0])
        for hh in range(8):
            als, ald = buf_pair[hh % 2][0], buf_pair[hh % 2][1]
            if hh < 7:
                dnext = issue(hh + 1, buf_pair[(hh + 1) % 2])
            dcur[0].wait()
            dcur[1].wait()
            for l4 in range(8):
                dv = plsc.load_gather(
                    dstc, [jnp.full((L,), hh, jnp.int32),
                           l4 * 16 + _iota16()])
                dvl = dv - nlo
                safe = (dvl >= 0) & (dvl < NH)
                plsc.store_scatter(
                    idxs, [_full16(0), l4 * 16 + _iota16()],
                    jnp.where(safe, dvl, 0))
                eid = (s * 10240 + g * 1024 + hh * 128
                       + l4 * 16 + _iota16())
                live = (eid < E) & safe
                er = l4 * 16 + _iota16()
                for h in range(H):
                    a = (plsc.load_gather(als, [er, _full16(h)])
                         + plsc.load_gather(ald, [er, _full16(H + h)]))
                    a = jnp.maximum(a, 0.2 * a)
                    ev = jnp.exp(a)
                    plsc.store_scatter(
                        evp, [l4 * 16 + _iota16(), _full16(h)],
                        jnp.where(live, ev, 0.0))
                    plsc.store_scatter(
                        evpk,
                        [_full16((hh % 2) * 4 + (l4 >> 1)),
                         _iota16() * 4 + ((l4 & 1) * 64 + h)],
                        jnp.where(eid < E, ev, 0.0))
            pltpu.sync_copy(evp, den_acc.at[idxs.at[0]], add=True)
            if hh % 2 == 1:
                w8 = (hh // 2) * 8

                @pl.when(c == 0)
                def _():
                    pltpu.sync_copy(
                        evpk,
                        ev_hbm.at[pl.ds(s * 320 + g * 32 + w8, 8)])
            if hh < 7:
                dcur = dnext

    plsc.subcore_barrier()
    pltpu.sync_copy(den_acc.at[pl.ds(s * 320, 320)],
                    den_hbm.at[c].at[pl.ds(s * 320, 320)])


def _phase1(alx, src128, dst128):
    return pl.kernel(
        _phase1_body,
        out_type=[
            jax.ShapeDtypeStruct((EVR, 128), jnp.float32),
            jax.ShapeDtypeStruct((NC, NH, 128), jnp.float32),
        ],
        mesh=_sc_mesh(),
        compiler_params=_sc_params(),
        scratch_types=[
            pltpu.VMEM((8, 128), jnp.int32),
            pltpu.VMEM((8, 128), jnp.int32),
            pltpu.VMEM((128, 128), jnp.float32),
            pltpu.VMEM((128, 128), jnp.float32),
            pltpu.VMEM((128, 128), jnp.float32),
            pltpu.VMEM((128, 128), jnp.float32),
            pltpu.VMEM((128, 128), jnp.float32),
            pltpu.VMEM((8, 128), jnp.float32),
            pltpu.VMEM((1, 128), jnp.int32),
            pltpu.VMEM((1, 64), jnp.int32),
            pltpu.SemaphoreType.DMA,
            pltpu.SemaphoreType.DMA,
            pltpu.SemaphoreType.DMA,
            pltpu.SemaphoreType.DMA,
            pltpu.VMEM_SHARED((NH, 128), jnp.float32),
        ],
    )(alx, src128, dst128)


def _rden_kernel(den_ref, rden_ref):
    rden_ref[...] = 0.25 / (den_ref[...] + 1e-16)


def _stage3(denomp):
    return pl.pallas_call(
        _rden_kernel,
        in_specs=[pl.BlockSpec((NC, NH, 128), lambda: (0, 0, 0))],
        out_specs=pl.BlockSpec((NC, NH, 128), lambda: (0, 0, 0)),
        out_shape=jax.ShapeDtypeStruct((NC, NH, 128), jnp.float32),
    )(denomp)


def _coef_body(ev_hbm, rden_hbm, dst_hbm, coef_hbm, dstc, rdbA, rdbB, evk,
               coefk, semA, semB):
    c = lax.axis_index("c")
    s = lax.axis_index("s")
    wid = c * NS + s

    @pl.loop(0, 5)
    def _(g):
        pltpu.sync_copy(dst_hbm.at[pl.ds(wid * 40 + g * 8, 8)], dstc)

        @pl.loop(0, 4)
        def _(w):
            d0 = pltpu.async_copy(rden_hbm.at[dstc.at[w * 2]], rdbA, semA)
            d1 = pltpu.async_copy(rden_hbm.at[dstc.at[w * 2 + 1]], rdbB, semB)
            pltpu.sync_copy(
                ev_hbm.at[pl.ds(wid * 160 + g * 32 + w * 8, 8)], evk)
            for half, (rdb, dd) in enumerate(((rdbA, d0), (rdbB, d1))):
                dd.wait()
                for l2 in range(8):
                    l = half * 8 + l2
                    for h in range(H):
                        rd = plsc.load_gather(
                            rdb, [l2 * 16 + _iota16(), _full16(h)])
                        lane = _iota16() * 4 + ((l & 1) * 64 + h)
                        evv = plsc.load_gather(evk, [_full16(l >> 1), lane])
                        plsc.store_scatter(coefk, [_full16(l >> 1), lane],
                                           evv * rd)
            pltpu.sync_copy(
                coefk, coef_hbm.at[pl.ds(wid * 160 + g * 32 + w * 8, 8)])


def _stage3b(ev, rden2d, dst128):
    return pl.kernel(
        _coef_body,
        out_type=jax.ShapeDtypeStruct((EVR, 128), jnp.float32),
        mesh=_sc_mesh(),
        compiler_params=_sc_params(),
        scratch_types=[
            pltpu.VMEM((8, 128), jnp.int32),
            pltpu.VMEM((128, 128), jnp.float32),
            pltpu.VMEM((128, 128), jnp.float32),
            pltpu.VMEM((8, 128), jnp.float32),
            pltpu.VMEM((8, 128), jnp.float32),
            pltpu.SemaphoreType.DMA,
            pltpu.SemaphoreType.DMA,
        ],
    )(ev, rden2d, dst128)


def _phase2_body(hg_hbm, coef_hbm, src_hbm, dst_hbm, oc_hbm,
                 srcc, dstc, coefc, igA, igB, idxsA, idxsB, idxz, grA, grB,
                 accbA, accbB, semA, semB, semSA, semSB, out_acc):
    c = lax.axis_index("c")
    s = lax.axis_index("s")

    zero16 = jnp.zeros((L,), jnp.float32)

    @pl.loop(0, 64)
    def _(r):
        for q in range(8):
            grA[r, pl.ds(q * 16, 16)] = zero16

    @pl.loop(0, 10)
    def _(z):
        for l in range(4):
            plsc.store_scatter(idxz, [_full16(0), l * 16 + _iota16()],
                               s * 640 + z * 64 + l * 16 + _iota16())
        pltpu.sync_copy(grA.at[pl.ds(0, 64)], out_acc.at[idxz.at[0]])
    plsc.subcore_barrier()

    @pl.loop(0, 10)
    def _(g):
        pltpu.sync_copy(src_hbm.at[pl.ds(s * 80 + g * 8, 8)], srcc)
        pltpu.sync_copy(dst_hbm.at[pl.ds(s * 80 + g * 8, 8)], dstc)

        @pl.loop(0, 4)
        def _(w):
            pltpu.sync_copy(
                coef_hbm.at[pl.ds(s * 320 + g * 32 + w * 8, 8)], coefc)

            def build_idx(k, ibuf):
                for l2 in range(2):
                    r = k * 32 + l2 * 16
                    sv = plsc.load_gather(
                        srcc, [jnp.full((L,), w * 2 + r // 128, jnp.int32),
                               (r % 128) + _iota16()])
                    for h in range(H):
                        plsc.store_scatter(
                            ibuf, [_full16(0), h * 32 + l2 * 16 + _iota16()],
                            sv + ((2 * h) * N + c * N))

            bufs = [(grA, igA, semA), (grB, igB, semB)]
            accs = [(accbA, idxsA, semSA), (accbB, idxsB, semSB)]
            dscat = [None, None]
            build_idx(0, bufs[0][1])
            dcur = pltpu.async_copy(hg_hbm.at[bufs[0][1].at[0]],
                                    bufs[0][0], bufs[0][2])
            for k in range(8):
                gr, ig, _sem = bufs[k % 2]
                ab, ai, asem = accs[k % 2]
                if k < 7:
                    gn, ign, semn = bufs[(k + 1) % 2]
                    build_idx(k + 1, ign)
                    dnext = pltpu.async_copy(hg_hbm.at[ign.at[0]], gn, semn)
                if dscat[k % 2] is not None:
                    dscat[k % 2].wait()
                    dscat[k % 2] = None
                for l2 in range(2):
                    r = k * 32 + l2 * 16
                    dv = plsc.load_gather(
                        dstc, [jnp.full((L,), w * 2 + r // 128, jnp.int32),
                               (r % 128) + _iota16()])
                    plsc.store_scatter(
                        ai, [_full16(0), l2 * 16 + _iota16()], dv)
                dcur.wait()

                @pl.loop(0, 32)
                def _(i):
                    p = (k * 32 + i) * 4
                    acc = [None] * 8
                    for h in range(H):
                        cb = plsc.load_gather(
                            coefc, [jnp.full((L,), (p + h) >> 7, jnp.int32),
                                    jnp.full((L,), (p + h) & 127, jnp.int32)])
                        for q in range(8):
                            r = gr[h * 32 + i, pl.ds(q * 16, 16)]
                            acc[q] = (r * cb if h == 0 else acc[q] + r * cb)
                    for q in range(8):
                        ab[i, pl.ds(q * 16, 16)] = acc[q]

                dscat[k % 2] = pltpu.async_copy(ab, out_acc.at[ai.at[0]],
                                                asem, add=True)
                if k < 7:
                    dcur = dnext
            for d in dscat:
                if d is not None:
                    d.wait()

    plsc.subcore_barrier()
    pltpu.sync_copy(out_acc.at[pl.ds(s * 640, 640)],
                    oc_hbm.at[c].at[pl.ds(s * 640, 640)])


def _phase2(hg, coef, src128, dst128):
    return pl.kernel(
        _phase2_body,
        out_type=jax.ShapeDtypeStruct((NC, RN, CH), jnp.float32),
        mesh=_sc_mesh(),
        compiler_params=_sc_params(),
        scratch_types=[
            pltpu.VMEM((8, 128), jnp.int32),
            pltpu.VMEM((8, 128), jnp.int32),
            pltpu.VMEM((8, 128), jnp.float32),
            pltpu.VMEM((1, 128), jnp.int32),
            pltpu.VMEM((1, 128), jnp.int32),
            pltpu.VMEM((1, 32), jnp.int32),
            pltpu.VMEM((1, 32), jnp.int32),
            pltpu.VMEM((1, 64), jnp.int32),
            pltpu.VMEM((128, CH), jnp.float32),
            pltpu.VMEM((128, CH), jnp.float32),
            pltpu.VMEM((32, CH), jnp.float32),
            pltpu.VMEM((32, CH), jnp.float32),
            pltpu.SemaphoreType.DMA,
            pltpu.SemaphoreType.DMA,
            pltpu.SemaphoreType.DMA,
            pltpu.SemaphoreType.DMA,
            pltpu.VMEM_SHARED((RN, CH), jnp.float32),
        ],
    )(hg, coef, src128, dst128)


def _final_kernel(oc_ref, b_ref, out_ref):
    out_ref[...] = (jnp.concatenate([oc_ref[0], oc_ref[1]], axis=1)
                    + b_ref[...])


def _stage5(oc, b2):
    return pl.pallas_call(
        _final_kernel,
        grid=(NBN,),
        in_specs=[
            pl.BlockSpec((NC, BN, CH), lambda nb: (0, nb, 0)),
            pl.BlockSpec((1, DOUT), lambda nb: (0, 0)),
        ],
        out_specs=pl.BlockSpec((BN, DOUT), lambda nb: (nb, 0)),
        out_shape=jax.ShapeDtypeStruct((N, DOUT), jnp.float32),
    )(oc, b2)


def kernel(x, edge_index, W, a_src, a_dst, b):
    src = edge_index[0]
    dst = edge_index[1]
    pad = EP - E
    srcp = jnp.concatenate([src, jnp.zeros((pad,), jnp.int32)]).reshape(ER, 128)
    dstp = jnp.concatenate([dst, jnp.zeros((pad,), jnp.int32)]).reshape(ER, 128)
    asr = a_src.reshape(NK, CH)
    adr = a_dst.reshape(NK, CH)

    hg, alx = _stage1(x, W, asr, adr)
    ev, denomp = _phase1(alx, srcp, dstp)
    rden = _stage3(denomp)
    coef = _stage3b(ev, rden.reshape(NC * NH, 128), dstp)
    oc = _phase2(hg, coef, srcp, dstp)
    return _stage5(oc, b.reshape(1, DOUT))
